# baseline (speedup 1.0000x reference)
import functools

import jax
import jax.numpy as jnp
from jax import lax
from jax.experimental import pallas as pl
from jax.experimental.pallas import tpu as pltpu
from jax.experimental.pallas import tpu_sc as plsc

N, E, D, H, RE, STEPS = 10000, 320000, 128, 256, 4, 4

NC, NS = 2, 16
NW = NC * NS
GW = 80
NP = 10240
RPT = NP // NS

BE = 2560
BN = 2000

EA = 153600
EB = E - EA


def _silu(v):
    return v * jax.nn.sigmoid(v)


def _ln_res(base, u, g, b):
    m = jnp.mean(u, axis=-1, keepdims=True)
    s = jnp.mean((u - m) ** 2, axis=-1, keepdims=True)
    return base + (u - m) / jnp.sqrt(s + 1e-5) * g + b


def _sc_gather(x, src, dst, epw, nwin):
    ne = epw * NW
    mesh = plsc.VectorSubcoreMesh(core_axis_name="c", subcore_axis_name="s")

    @functools.partial(
        pl.kernel,
        mesh=mesh,
        out_type=(jax.ShapeDtypeStruct((ne, D), jnp.float32),
                  jax.ShapeDtypeStruct((ne, D), jnp.float32)),
        scratch_types=[pltpu.VMEM((GW,), jnp.int32)] * 4
                      + [pltpu.VMEM((GW, D), jnp.float32)] * 4
                      + [pltpu.VMEM_SHARED((N, D), jnp.float32)]
                      + [pltpu.SemaphoreType.DMA] * 12,
    )
    def k(x_hbm, src_hbm, dst_hbm, xs_hbm, xd_hbm,
          is0, is1, id0, id1, rs0, rs1, rd0, rd1, xsh, *sems):
        idx = ((is0, is1), (id0, id1))
        row = ((rs0, rs1), (rd0, rd1))
        sem_i = ((sems[0], sems[1]), (sems[2], sems[3]))
        sem_g = ((sems[4], sems[5]), (sems[6], sems[7]))
        sem_o = ((sems[8], sems[9]), (sems[10], sems[11]))
        ins = (src_hbm, dst_hbm)
        outs = (xs_hbm, xd_hbm)
        sid = lax.axis_index("s")
        wid = sid * NC + lax.axis_index("c")
        base = wid * epw

        tail = N - (NS - 1) * RPT

        @pl.when(sid < NS - 1)
        def _():
            pltpu.sync_copy(x_hbm.at[pl.ds(sid * RPT, RPT)],
                            xsh.at[pl.ds(sid * RPT, RPT)])

        @pl.when(sid == NS - 1)
        def _():
            pltpu.sync_copy(x_hbm.at[pl.ds((NS - 1) * RPT, tail)],
                            xsh.at[pl.ds((NS - 1) * RPT, tail)])

        plsc.subcore_barrier()

        def emit(w, p):
            q = 1 - p

            @pl.when((w >= 2) & (w < nwin + 2))
            def _():
                for sd in range(2):
                    pltpu.make_async_copy(
                        xsh.at[idx[sd][p]], row[sd][p], sem_g[sd][p]).wait()
                    pltpu.async_copy(
                        row[sd][p],
                        outs[sd].at[pl.ds(base + (w - 2) * GW, GW)],
                        sem_o[sd][p])

            @pl.when((w >= 3) & (w < nwin + 3))
            def _():
                for sd in range(2):
                    pltpu.make_async_copy(
                        row[sd][q],
                        outs[sd].at[pl.ds(base + (w - 3) * GW, GW)],
                        sem_o[sd][q]).wait()

            @pl.when((w >= 1) & (w < nwin + 1))
            def _():
                for sd in range(2):
                    pltpu.make_async_copy(
                        ins[sd].at[pl.ds(base + (w - 1) * GW, GW)],
                        idx[sd][q], sem_i[sd][q]).wait()
                    pltpu.async_copy(
                        xsh.at[idx[sd][q]], row[sd][q], sem_g[sd][q])

            @pl.when(w < nwin)
            def _():
                for sd in range(2):
                    pltpu.async_copy(
                        ins[sd].at[pl.ds(base + w * GW, GW)],
                        idx[sd][p], sem_i[sd][p])

        def body(kk, carry):
            emit(2 * kk, 0)
            emit(2 * kk + 1, 1)
            return carry

        lax.fori_loop(0, (nwin + 4) // 2, body, 0)

    return k(x, src, dst)


def _sc_scatter(ea, dst, zeros, epw, nwin, e0):
    ne = epw * NW
    epc = ne // NC
    mesh = plsc.VectorSubcoreMesh(core_axis_name="c", subcore_axis_name="s")

    @functools.partial(
        pl.kernel,
        mesh=mesh,
        out_type=jax.ShapeDtypeStruct((NC, NP, D), jnp.float32),
        scratch_types=[pltpu.VMEM((GW,), jnp.int32)] * 2
                      + [pltpu.VMEM((GW, D), jnp.float32)] * 2
                      + [pltpu.VMEM_SHARED((NP, D), jnp.float32)]
                      + [pltpu.SemaphoreType.DMA] * 6,
    )
    def k(ea_hbm, dst_hbm, z_hbm, out_hbm,
          ix0, ix1, rw0, rw1, acc, *sems):
        idx = (ix0, ix1)
        row = (rw0, rw1)
        sem_ai = (sems[0], sems[1])
        sem_ar = (sems[2], sems[3])
        sem_b = (sems[4], sems[5])
        cid = lax.axis_index("c")
        sid = lax.axis_index("s")
        r0 = sid * RPT
        pltpu.sync_copy(z_hbm.at[pl.ds(r0, RPT)], acc.at[pl.ds(r0, RPT)])
        plsc.subcore_barrier()
        base = cid * epc + sid * epw

        def emit(w, p):
            q = 1 - p

            @pl.when((w >= 2) & (w < nwin + 2))
            def _():
                pltpu.make_async_copy(row[p], acc.at[idx[p]], sem_b[p]).wait()

            @pl.when(w < nwin)
            def _():
                off = base + w * GW
                pltpu.async_copy(dst_hbm.at[pl.ds(off + e0, GW)], idx[p],
                                 sem_ai[p])
                pltpu.async_copy(ea_hbm.at[pl.ds(off, GW)], row[p],
                                 sem_ar[p])

            @pl.when((w >= 1) & (w < nwin + 1))
            def _():
                off = base + (w - 1) * GW
                pltpu.make_async_copy(dst_hbm.at[pl.ds(off + e0, GW)], idx[q],
                                      sem_ai[q]).wait()
                pltpu.make_async_copy(ea_hbm.at[pl.ds(off, GW)], row[q],
                                      sem_ar[q]).wait()
                pltpu.async_copy(row[q], acc.at[idx[q]], sem_b[q], add=True)

        def body(kk, carry):
            emit(2 * kk, 0)
            emit(2 * kk + 1, 1)
            return carry

        lax.fori_loop(0, (nwin + 4) // 2, body, 0)
        plsc.subcore_barrier()
        pltpu.sync_copy(acc.at[pl.ds(r0, RPT)], out_hbm.at[cid, pl.ds(r0, RPT)])

    return k(ea, dst, zeros)


def _tc_edge(first, ne, eblk, ea_or_raw, xs, xd, W_enc, b_enc,
             Wea, Wes, Wed, be1, We2, be2, g_e, b_e):
    in_d = RE if first else D

    def body(ea_ref, xs_ref, xd_ref, wenc_ref, benc_ref, wea_ref, wes_ref,
             wed_ref, be1_ref, we2_ref, be2_ref, ge_ref, be_ref, out_ref):
        if first:
            ea = _silu(
                jnp.dot(ea_ref[...], wenc_ref[...],
                        preferred_element_type=jnp.float32) + benc_ref[...])
        else:
            ea = ea_ref[...]
        xs_bf = xs_ref[...].astype(jnp.bfloat16)
        xd_bf = xd_ref[...].astype(jnp.bfloat16)
        h = (jnp.dot(ea.astype(jnp.bfloat16), wea_ref[...],
                     preferred_element_type=jnp.float32)
             + jnp.dot(xs_bf, wes_ref[...],
                       preferred_element_type=jnp.float32)
             + jnp.dot(xd_bf, wed_ref[...],
                       preferred_element_type=jnp.float32)
             + be1_ref[...])
        h = _silu(h)
        u = jnp.dot(h.astype(jnp.bfloat16), we2_ref[...],
                    preferred_element_type=jnp.float32) + be2_ref[...]
        out_ref[...] = _ln_res(ea, u, ge_ref[...], be_ref[...])

    full = lambda shape: pl.BlockSpec(shape, lambda i: (0, 0))
    return pl.pallas_call(
        body,
        grid=(ne // BE,),
        in_specs=[
            pl.BlockSpec((BE, in_d), lambda i: (i + (eblk if first else 0), 0)),
            pl.BlockSpec((BE, D), lambda i: (i + eblk, 0)),
            pl.BlockSpec((BE, D), lambda i: (i + eblk, 0)),
            full((RE, D)), full((1, D)),
            full((D, H)), full((D, H)), full((D, H)), full((1, H)),
            full((H, D)), full((1, D)),
            full((1, D)), full((1, D)),
        ],
        out_specs=pl.BlockSpec((BE, D), lambda i: (i, 0)),
        out_shape=jax.ShapeDtypeStruct((ne, D), jnp.float32),
    )(ea_or_raw, xs, xd, W_enc, b_enc, Wea, Wes, Wed, be1, We2, be2, g_e, b_e)


def _tc_node(x, parts, Wna, Wnb, bn1, Wn2, bn2, g_n, b_n):
    nparts = len(parts)

    def body(x_ref, *refs):
        p_refs = refs[:nparts]
        (wna_ref, wnb_ref, bn1_ref, wn2_ref, bn2_ref, gn_ref, bn_ref,
         out_ref) = refs[nparts:]
        xb = x_ref[...]
        agg = p_refs[0][...]
        for pr in p_refs[1:]:
            agg = agg + pr[...]
        h = (jnp.dot(xb, wna_ref[...], preferred_element_type=jnp.float32)
             + jnp.dot(agg, wnb_ref[...], preferred_element_type=jnp.float32)
             + bn1_ref[...])
        h = _silu(h)
        u = jnp.dot(h, wn2_ref[...],
                    preferred_element_type=jnp.float32) + bn2_ref[...]
        out_ref[...] = _ln_res(xb, u, gn_ref[...], bn_ref[...])

    full = lambda shape: pl.BlockSpec(shape, lambda i: (0, 0))
    return pl.pallas_call(
        body,
        grid=(N // BN,),
        in_specs=[pl.BlockSpec((BN, D), lambda i: (i, 0))]
                 + [pl.BlockSpec((BN, D), lambda i: (i, 0))] * nparts
                 + [full((D, H)), full((D, H)), full((1, H)),
                    full((H, D)), full((1, D)),
                    full((1, D)), full((1, D))],
        out_specs=pl.BlockSpec((BN, D), lambda i: (i, 0)),
        out_shape=jax.ShapeDtypeStruct((N, D), jnp.float32),
    )(x, *parts, Wna, Wnb, bn1, Wn2, bn2, g_n, b_n)


def kernel(x, edge_index, edge_attr_raw, W_enc, b_enc, We1, be1, We2, be2,
           g_e, b_e, Wn1, bn1, Wn2, bn2, g_n, b_n):
    src = edge_index[0].astype(jnp.int32)
    dst = edge_index[1].astype(jnp.int32)
    zeros = jnp.zeros((NP, D), jnp.float32)

    Wea = We1[:D].astype(jnp.bfloat16)
    Wes = We1[D:2 * D].astype(jnp.bfloat16)
    Wed = We1[2 * D:].astype(jnp.bfloat16)
    We2b = We2.astype(jnp.bfloat16)
    Wna, Wnb = Wn1[:D], Wn1[D:]
    b_enc2 = b_enc.reshape(1, D)
    be1_2, be2_2 = be1.reshape(1, H), be2.reshape(1, D)
    ge2, be_2 = g_e.reshape(1, D), b_e.reshape(1, D)
    bn1_2, bn2_2 = bn1.reshape(1, H), bn2.reshape(1, D)
    gn2, bn_2 = g_n.reshape(1, D), b_n.reshape(1, D)

    halves = [
        dict(e0=0, ne=EA, epw=EA // NW, nwin=EA // NW // GW,
             ea=edge_attr_raw),
        dict(e0=EA, ne=EB, epw=EB // NW, nwin=EB // NW // GW,
             ea=edge_attr_raw),
    ]
    for step in range(STEPS):
        xs, xd = _sc_gather(x, src, dst, E // NW, E // NW // GW)
        for hv in halves:
            hv["ea"] = _tc_edge(step == 0, hv["ne"], hv["e0"] // BE,
                                hv["ea"], xs, xd,
                                W_enc, b_enc2, Wea, Wes, Wed, be1_2,
                                We2b, be2_2, ge2, be_2)
        for hv in halves:
            hv["p"] = _sc_scatter(hv["ea"], dst, zeros,
                                  hv["epw"], hv["nwin"], hv["e0"])
        parts = [hv["p"][c, :N] for hv in halves for c in range(NC)]
        x = _tc_node(x, parts, Wna, Wnb, bn1_2, Wn2, bn2_2, gn2, bn_2)
    return x


# speedup vs baseline: 1.0520x; 1.0520x over previous
import functools

import jax
import jax.numpy as jnp
from jax import lax
from jax.experimental import pallas as pl
from jax.experimental.pallas import tpu as pltpu
from jax.experimental.pallas import tpu_sc as plsc

N, E, D, H, RE, STEPS = 10000, 320000, 128, 256, 4, 4

NC, NS = 2, 16
NW = NC * NS
GW = 80
NP = 10240
RPT = NP // NS

BE = 2560
BN = 2000

EA = 153600
EB = E - EA


def _silu(v):
    return v * jax.nn.sigmoid(v)


def _ln_res(base, u, g, b):
    m = jnp.mean(u, axis=-1, keepdims=True)
    s = jnp.mean((u - m) ** 2, axis=-1, keepdims=True)
    return base + (u - m) / jnp.sqrt(s + 1e-5) * g + b


def _sc_gather(x, src, dst, epw, nwin):
    ne = epw * NW
    mesh = plsc.VectorSubcoreMesh(core_axis_name="c", subcore_axis_name="s")

    @functools.partial(
        pl.kernel,
        mesh=mesh,
        out_type=(jax.ShapeDtypeStruct((ne, D), jnp.float32),
                  jax.ShapeDtypeStruct((ne, D), jnp.float32)),
        scratch_types=[pltpu.VMEM((GW,), jnp.int32)] * 4
                      + [pltpu.VMEM((GW, D), jnp.float32)] * 4
                      + [pltpu.VMEM_SHARED((N, D), jnp.float32)]
                      + [pltpu.SemaphoreType.DMA] * 12,
    )
    def k(x_hbm, src_hbm, dst_hbm, xs_hbm, xd_hbm,
          is0, is1, id0, id1, rs0, rs1, rd0, rd1, xsh, *sems):
        idx = ((is0, is1), (id0, id1))
        row = ((rs0, rs1), (rd0, rd1))
        sem_i = ((sems[0], sems[1]), (sems[2], sems[3]))
        sem_g = ((sems[4], sems[5]), (sems[6], sems[7]))
        sem_o = ((sems[8], sems[9]), (sems[10], sems[11]))
        ins = (src_hbm, dst_hbm)
        outs = (xs_hbm, xd_hbm)
        sid = lax.axis_index("s")
        wid = sid * NC + lax.axis_index("c")
        base = wid * epw

        tail = N - (NS - 1) * RPT

        @pl.when(sid < NS - 1)
        def _():
            pltpu.sync_copy(x_hbm.at[pl.ds(sid * RPT, RPT)],
                            xsh.at[pl.ds(sid * RPT, RPT)])

        @pl.when(sid == NS - 1)
        def _():
            pltpu.sync_copy(x_hbm.at[pl.ds((NS - 1) * RPT, tail)],
                            xsh.at[pl.ds((NS - 1) * RPT, tail)])

        plsc.subcore_barrier()

        def emit(w, p):
            q = 1 - p

            @pl.when((w >= 2) & (w < nwin + 2))
            def _():
                for sd in range(2):
                    pltpu.make_async_copy(
                        xsh.at[idx[sd][p]], row[sd][p], sem_g[sd][p]).wait()
                    pltpu.async_copy(
                        row[sd][p],
                        outs[sd].at[pl.ds(base + (w - 2) * GW, GW)],
                        sem_o[sd][p])

            @pl.when((w >= 3) & (w < nwin + 3))
            def _():
                for sd in range(2):
                    pltpu.make_async_copy(
                        row[sd][q],
                        outs[sd].at[pl.ds(base + (w - 3) * GW, GW)],
                        sem_o[sd][q]).wait()

            @pl.when((w >= 1) & (w < nwin + 1))
            def _():
                for sd in range(2):
                    pltpu.make_async_copy(
                        ins[sd].at[pl.ds(base + (w - 1) * GW, GW)],
                        idx[sd][q], sem_i[sd][q]).wait()
                    pltpu.async_copy(
                        xsh.at[idx[sd][q]], row[sd][q], sem_g[sd][q])

            @pl.when(w < nwin)
            def _():
                for sd in range(2):
                    pltpu.async_copy(
                        ins[sd].at[pl.ds(base + w * GW, GW)],
                        idx[sd][p], sem_i[sd][p])

        def body(kk, carry):
            emit(2 * kk, 0)
            emit(2 * kk + 1, 1)
            return carry

        lax.fori_loop(0, (nwin + 4) // 2, body, 0)

    return k(x, src, dst)


def _sc_scatter(ea, dst, zeros, epw, nwin, e0):
    ne = epw * NW
    epc = ne // NC
    mesh = plsc.VectorSubcoreMesh(core_axis_name="c", subcore_axis_name="s")

    @functools.partial(
        pl.kernel,
        mesh=mesh,
        out_type=jax.ShapeDtypeStruct((NC, NP, D), jnp.float32),
        scratch_types=[pltpu.VMEM((GW,), jnp.int32)] * 2
                      + [pltpu.VMEM((GW, D), jnp.float32)] * 2
                      + [pltpu.VMEM_SHARED((NP, D), jnp.float32)]
                      + [pltpu.SemaphoreType.DMA] * 6,
    )
    def k(ea_hbm, dst_hbm, z_hbm, out_hbm,
          ix0, ix1, rw0, rw1, acc, *sems):
        idx = (ix0, ix1)
        row = (rw0, rw1)
        sem_ai = (sems[0], sems[1])
        sem_ar = (sems[2], sems[3])
        sem_b = (sems[4], sems[5])
        cid = lax.axis_index("c")
        sid = lax.axis_index("s")
        r0 = sid * RPT
        pltpu.sync_copy(z_hbm.at[pl.ds(r0, RPT)], acc.at[pl.ds(r0, RPT)])
        plsc.subcore_barrier()
        base = cid * epc + sid * epw

        def emit(w, p):
            q = 1 - p

            @pl.when((w >= 2) & (w < nwin + 2))
            def _():
                pltpu.make_async_copy(row[p], acc.at[idx[p]], sem_b[p]).wait()

            @pl.when(w < nwin)
            def _():
                off = base + w * GW
                pltpu.async_copy(dst_hbm.at[pl.ds(off + e0, GW)], idx[p],
                                 sem_ai[p])
                pltpu.async_copy(ea_hbm.at[pl.ds(off, GW)], row[p],
                                 sem_ar[p])

            @pl.when((w >= 1) & (w < nwin + 1))
            def _():
                off = base + (w - 1) * GW
                pltpu.make_async_copy(dst_hbm.at[pl.ds(off + e0, GW)], idx[q],
                                      sem_ai[q]).wait()
                pltpu.make_async_copy(ea_hbm.at[pl.ds(off, GW)], row[q],
                                      sem_ar[q]).wait()
                pltpu.async_copy(row[q], acc.at[idx[q]], sem_b[q], add=True)

        def body(kk, carry):
            emit(2 * kk, 0)
            emit(2 * kk + 1, 1)
            return carry

        lax.fori_loop(0, (nwin + 4) // 2, body, 0)
        plsc.subcore_barrier()
        pltpu.sync_copy(acc.at[pl.ds(r0, RPT)], out_hbm.at[cid, pl.ds(r0, RPT)])

    return k(ea, dst, zeros)


def _tc_edge(first, ea_or_raw, xs, xd, W_enc, b_enc,
             Wea, Wes, Wed, be1, We2, be2, g_e, b_e):
    ne = xs.shape[0]
    in_d = RE if first else D

    def body(ea_ref, xs_ref, xd_ref, wenc_ref, benc_ref, wea_ref, wes_ref,
             wed_ref, be1_ref, we2_ref, be2_ref, ge_ref, be_ref, out_ref):
        if first:
            ea = _silu(
                jnp.dot(ea_ref[...], wenc_ref[...],
                        preferred_element_type=jnp.float32) + benc_ref[...])
        else:
            ea = ea_ref[...]
        xs_bf = xs_ref[...].astype(jnp.bfloat16)
        xd_bf = xd_ref[...].astype(jnp.bfloat16)
        h = (jnp.dot(ea.astype(jnp.bfloat16), wea_ref[...],
                     preferred_element_type=jnp.float32)
             + jnp.dot(xs_bf, wes_ref[...],
                       preferred_element_type=jnp.float32)
             + jnp.dot(xd_bf, wed_ref[...],
                       preferred_element_type=jnp.float32)
             + be1_ref[...])
        h = _silu(h)
        u = jnp.dot(h.astype(jnp.bfloat16), we2_ref[...],
                    preferred_element_type=jnp.float32) + be2_ref[...]
        out_ref[...] = _ln_res(ea, u, ge_ref[...], be_ref[...])

    full = lambda shape: pl.BlockSpec(shape, lambda i: (0, 0))
    return pl.pallas_call(
        body,
        grid=(ne // BE,),
        in_specs=[
            pl.BlockSpec((BE, in_d), lambda i: (i, 0)),
            pl.BlockSpec((BE, D), lambda i: (i, 0)),
            pl.BlockSpec((BE, D), lambda i: (i, 0)),
            full((RE, D)), full((1, D)),
            full((D, H)), full((D, H)), full((D, H)), full((1, H)),
            full((H, D)), full((1, D)),
            full((1, D)), full((1, D)),
        ],
        out_specs=pl.BlockSpec((BE, D), lambda i: (i, 0)),
        out_shape=jax.ShapeDtypeStruct((ne, D), jnp.float32),
    )(ea_or_raw, xs, xd, W_enc, b_enc, Wea, Wes, Wed, be1, We2, be2, g_e, b_e)


def _tc_node(x, parts, Wna, Wnb, bn1, Wn2, bn2, g_n, b_n):
    nparts = len(parts)

    def body(x_ref, *refs):
        p_refs = refs[:nparts]
        (wna_ref, wnb_ref, bn1_ref, wn2_ref, bn2_ref, gn_ref, bn_ref,
         out_ref) = refs[nparts:]
        xb = x_ref[...]
        agg = p_refs[0][...]
        for pr in p_refs[1:]:
            agg = agg + pr[...]
        h = (jnp.dot(xb, wna_ref[...], preferred_element_type=jnp.float32)
             + jnp.dot(agg, wnb_ref[...], preferred_element_type=jnp.float32)
             + bn1_ref[...])
        h = _silu(h)
        u = jnp.dot(h, wn2_ref[...],
                    preferred_element_type=jnp.float32) + bn2_ref[...]
        out_ref[...] = _ln_res(xb, u, gn_ref[...], bn_ref[...])

    full = lambda shape: pl.BlockSpec(shape, lambda i: (0, 0))
    return pl.pallas_call(
        body,
        grid=(N // BN,),
        in_specs=[pl.BlockSpec((BN, D), lambda i: (i, 0))]
                 + [pl.BlockSpec((BN, D), lambda i: (i, 0))] * nparts
                 + [full((D, H)), full((D, H)), full((1, H)),
                    full((H, D)), full((1, D)),
                    full((1, D)), full((1, D))],
        out_specs=pl.BlockSpec((BN, D), lambda i: (i, 0)),
        out_shape=jax.ShapeDtypeStruct((N, D), jnp.float32),
    )(x, *parts, Wna, Wnb, bn1, Wn2, bn2, g_n, b_n)


def kernel(x, edge_index, edge_attr_raw, W_enc, b_enc, We1, be1, We2, be2,
           g_e, b_e, Wn1, bn1, Wn2, bn2, g_n, b_n):
    src = edge_index[0].astype(jnp.int32)
    dst = edge_index[1].astype(jnp.int32)
    zeros = jnp.zeros((NP, D), jnp.float32)

    Wea = We1[:D].astype(jnp.bfloat16)
    Wes = We1[D:2 * D].astype(jnp.bfloat16)
    Wed = We1[2 * D:].astype(jnp.bfloat16)
    We2b = We2.astype(jnp.bfloat16)
    Wna, Wnb = Wn1[:D], Wn1[D:]
    b_enc2 = b_enc.reshape(1, D)
    be1_2, be2_2 = be1.reshape(1, H), be2.reshape(1, D)
    ge2, be_2 = g_e.reshape(1, D), b_e.reshape(1, D)
    bn1_2, bn2_2 = bn1.reshape(1, H), bn2.reshape(1, D)
    gn2, bn_2 = g_n.reshape(1, D), b_n.reshape(1, D)

    halves = [
        dict(e0=0, src=src[:EA], dst4=dst[:EA], ea=edge_attr_raw[:EA],
             epw=EA // NW, nwin=EA // NW // GW),
        dict(e0=EA, src=src[EA:], dst4=dst[EA:], ea=edge_attr_raw[EA:],
             epw=EB // NW, nwin=EB // NW // GW),
    ]
    for step in range(STEPS):
        for hv in halves:
            hv["xs"], hv["xd"] = _sc_gather(x, hv["src"], hv["dst4"],
                                            hv["epw"], hv["nwin"])
        for hv in halves:
            hv["ea"] = _tc_edge(step == 0, hv["ea"], hv["xs"], hv["xd"],
                                W_enc, b_enc2, Wea, Wes, Wed, be1_2,
                                We2b, be2_2, ge2, be_2)
        for hv in halves:
            hv["p"] = _sc_scatter(hv["ea"], dst, zeros,
                                  hv["epw"], hv["nwin"], hv["e0"])
        parts = [hv["p"][c, :N] for hv in halves for c in range(NC)]
        x = _tc_node(x, parts, Wna, Wnb, bn1_2, Wn2, bn2_2, gn2, bn_2)
    return x


# speedup vs baseline: 1.0747x; 1.0216x over previous
import functools

import jax
import jax.numpy as jnp
from jax import lax
from jax.experimental import pallas as pl
from jax.experimental.pallas import tpu as pltpu
from jax.experimental.pallas import tpu_sc as plsc

N, E, D, H, RE, STEPS = 10000, 320000, 128, 256, 4, 4

NC, NS = 2, 16
NW = NC * NS
GW = 80
NP = 10240
RPT = NP // NS

BE = 2560
BN = 2000

EA = 153600
EB = E - EA


def _silu(v):
    return v * jax.nn.sigmoid(v)


def _ln_res(base, u, g, b):
    m = jnp.mean(u, axis=-1, keepdims=True)
    s = jnp.mean((u - m) ** 2, axis=-1, keepdims=True)
    return base + (u - m) / jnp.sqrt(s + 1e-5) * g + b


def _sc_gather(x, src, dst, epw, nwin):
    ne = epw * NW
    mesh = plsc.VectorSubcoreMesh(core_axis_name="c", subcore_axis_name="s")

    @functools.partial(
        pl.kernel,
        mesh=mesh,
        out_type=(jax.ShapeDtypeStruct((ne, D), jnp.float32),
                  jax.ShapeDtypeStruct((ne, D), jnp.float32)),
        scratch_types=[pltpu.VMEM((GW,), jnp.int32)] * 4
                      + [pltpu.VMEM((GW, D), jnp.float32)] * 4
                      + [pltpu.VMEM_SHARED((N, D), jnp.float32)]
                      + [pltpu.SemaphoreType.DMA] * 12,
    )
    def k(x_hbm, src_hbm, dst_hbm, xs_hbm, xd_hbm,
          is0, is1, id0, id1, rs0, rs1, rd0, rd1, xsh, *sems):
        idx = ((is0, is1), (id0, id1))
        row = ((rs0, rs1), (rd0, rd1))
        sem_i = ((sems[0], sems[1]), (sems[2], sems[3]))
        sem_g = ((sems[4], sems[5]), (sems[6], sems[7]))
        sem_o = ((sems[8], sems[9]), (sems[10], sems[11]))
        ins = (src_hbm, dst_hbm)
        outs = (xs_hbm, xd_hbm)
        sid = lax.axis_index("s")
        wid = sid * NC + lax.axis_index("c")
        base = wid * epw

        tail = N - (NS - 1) * RPT

        @pl.when(sid < NS - 1)
        def _():
            pltpu.sync_copy(x_hbm.at[pl.ds(sid * RPT, RPT)],
                            xsh.at[pl.ds(sid * RPT, RPT)])

        @pl.when(sid == NS - 1)
        def _():
            pltpu.sync_copy(x_hbm.at[pl.ds((NS - 1) * RPT, tail)],
                            xsh.at[pl.ds((NS - 1) * RPT, tail)])

        plsc.subcore_barrier()

        def emit(w, p):
            q = 1 - p

            @pl.when((w >= 2) & (w < nwin + 2))
            def _():
                for sd in range(2):
                    pltpu.make_async_copy(
                        xsh.at[idx[sd][p]], row[sd][p], sem_g[sd][p]).wait()
                    pltpu.async_copy(
                        row[sd][p],
                        outs[sd].at[pl.ds(base + (w - 2) * GW, GW)],
                        sem_o[sd][p])

            @pl.when((w >= 3) & (w < nwin + 3))
            def _():
                for sd in range(2):
                    pltpu.make_async_copy(
                        row[sd][q],
                        outs[sd].at[pl.ds(base + (w - 3) * GW, GW)],
                        sem_o[sd][q]).wait()

            @pl.when((w >= 1) & (w < nwin + 1))
            def _():
                for sd in range(2):
                    pltpu.make_async_copy(
                        ins[sd].at[pl.ds(base + (w - 1) * GW, GW)],
                        idx[sd][q], sem_i[sd][q]).wait()
                    pltpu.async_copy(
                        xsh.at[idx[sd][q]], row[sd][q], sem_g[sd][q])

            @pl.when(w < nwin)
            def _():
                for sd in range(2):
                    pltpu.async_copy(
                        ins[sd].at[pl.ds(base + w * GW, GW)],
                        idx[sd][p], sem_i[sd][p])

        def body(kk, carry):
            emit(2 * kk, 0)
            emit(2 * kk + 1, 1)
            return carry

        lax.fori_loop(0, (nwin + 4) // 2, body, 0)

    return k(x, src, dst)


def _sc_scatter(ea, dst, zeros, epw, nwin, e0):
    ne = epw * NW
    epc = ne // NC
    mesh = plsc.VectorSubcoreMesh(core_axis_name="c", subcore_axis_name="s")

    @functools.partial(
        pl.kernel,
        mesh=mesh,
        out_type=jax.ShapeDtypeStruct((NC, NP, D), jnp.float32),
        scratch_types=[pltpu.VMEM((GW,), jnp.int32)] * 2
                      + [pltpu.VMEM((GW, D), jnp.float32)] * 2
                      + [pltpu.VMEM_SHARED((NP, D), jnp.float32)]
                      + [pltpu.SemaphoreType.DMA] * 6,
    )
    def k(ea_hbm, dst_hbm, z_hbm, out_hbm,
          ix0, ix1, rw0, rw1, acc, *sems):
        idx = (ix0, ix1)
        row = (rw0, rw1)
        sem_ai = (sems[0], sems[1])
        sem_ar = (sems[2], sems[3])
        sem_b = (sems[4], sems[5])
        cid = lax.axis_index("c")
        sid = lax.axis_index("s")
        r0 = sid * RPT
        pltpu.sync_copy(z_hbm.at[pl.ds(r0, RPT)], acc.at[pl.ds(r0, RPT)])
        plsc.subcore_barrier()
        base = cid * epc + sid * epw

        def emit(w, p):
            q = 1 - p

            @pl.when((w >= 2) & (w < nwin + 2))
            def _():
                pltpu.make_async_copy(row[p], acc.at[idx[p]], sem_b[p]).wait()

            @pl.when(w < nwin)
            def _():
                off = base + w * GW
                pltpu.async_copy(dst_hbm.at[pl.ds(off + e0, GW)], idx[p],
                                 sem_ai[p])
                pltpu.async_copy(ea_hbm.at[pl.ds(off, GW)], row[p],
                                 sem_ar[p])

            @pl.when((w >= 1) & (w < nwin + 1))
            def _():
                off = base + (w - 1) * GW
                pltpu.make_async_copy(dst_hbm.at[pl.ds(off + e0, GW)], idx[q],
                                      sem_ai[q]).wait()
                pltpu.make_async_copy(ea_hbm.at[pl.ds(off, GW)], row[q],
                                      sem_ar[q]).wait()
                pltpu.async_copy(row[q], acc.at[idx[q]], sem_b[q], add=True)

        def body(kk, carry):
            emit(2 * kk, 0)
            emit(2 * kk + 1, 1)
            return carry

        lax.fori_loop(0, (nwin + 4) // 2, body, 0)
        plsc.subcore_barrier()
        pltpu.sync_copy(acc.at[pl.ds(r0, RPT)], out_hbm.at[cid, pl.ds(r0, RPT)])

    return k(ea, dst, zeros)


def _tc_edge(first, ea_or_raw, xs, xd, W_enc, b_enc,
             Wea, Wes, Wed, be1, We2, be2, g_e, b_e):
    ne = xs.shape[0]
    in_d = RE if first else D

    def body(ea_ref, xs_ref, xd_ref, wenc_ref, benc_ref, wea_ref, wes_ref,
             wed_ref, be1_ref, we2_ref, be2_ref, ge_ref, be_ref, out_ref):
        if first:
            ea = _silu(
                jnp.dot(ea_ref[...], wenc_ref[...],
                        preferred_element_type=jnp.float32) + benc_ref[...])
        else:
            ea = ea_ref[...]
        xs_bf = xs_ref[...].astype(jnp.bfloat16)
        xd_bf = xd_ref[...].astype(jnp.bfloat16)
        h = (jnp.dot(ea.astype(jnp.bfloat16), wea_ref[...],
                     preferred_element_type=jnp.float32)
             + jnp.dot(xs_bf, wes_ref[...],
                       preferred_element_type=jnp.float32)
             + jnp.dot(xd_bf, wed_ref[...],
                       preferred_element_type=jnp.float32)
             + be1_ref[...])
        h = _silu(h)
        u = jnp.dot(h.astype(jnp.bfloat16), we2_ref[...],
                    preferred_element_type=jnp.float32) + be2_ref[...]
        out_ref[...] = _ln_res(ea, u, ge_ref[...], be_ref[...])

    full = lambda shape: pl.BlockSpec(shape, lambda i: (0, 0))
    return pl.pallas_call(
        body,
        grid=(ne // BE,),
        in_specs=[
            pl.BlockSpec((BE, in_d), lambda i: (i, 0)),
            pl.BlockSpec((BE, D), lambda i: (i, 0)),
            pl.BlockSpec((BE, D), lambda i: (i, 0)),
            full((RE, D)), full((1, D)),
            full((D, H)), full((D, H)), full((D, H)), full((1, H)),
            full((H, D)), full((1, D)),
            full((1, D)), full((1, D)),
        ],
        out_specs=pl.BlockSpec((BE, D), lambda i: (i, 0)),
        out_shape=jax.ShapeDtypeStruct((ne, D), jnp.float32),
    )(ea_or_raw, xs, xd, W_enc, b_enc, Wea, Wes, Wed, be1, We2, be2, g_e, b_e)


def _tc_node(x, parts, Wna, Wnb, bn1, Wn2, bn2, g_n, b_n):
    nparts = len(parts)

    def body(x_ref, *refs):
        p_refs = refs[:nparts]
        (wna_ref, wnb_ref, bn1_ref, wn2_ref, bn2_ref, gn_ref, bn_ref,
         out_ref) = refs[nparts:]
        xb = x_ref[...]
        agg = p_refs[0][...]
        for pr in p_refs[1:]:
            agg = agg + pr[...]
        h = (jnp.dot(xb, wna_ref[...], preferred_element_type=jnp.float32)
             + jnp.dot(agg, wnb_ref[...], preferred_element_type=jnp.float32)
             + bn1_ref[...])
        h = _silu(h)
        u = jnp.dot(h, wn2_ref[...],
                    preferred_element_type=jnp.float32) + bn2_ref[...]
        out_ref[...] = _ln_res(xb, u, gn_ref[...], bn_ref[...])

    full = lambda shape: pl.BlockSpec(shape, lambda i: (0, 0))
    return pl.pallas_call(
        body,
        grid=(N // BN,),
        in_specs=[pl.BlockSpec((BN, D), lambda i: (i, 0))]
                 + [pl.BlockSpec((BN, D), lambda i: (i, 0))] * nparts
                 + [full((D, H)), full((D, H)), full((1, H)),
                    full((H, D)), full((1, D)),
                    full((1, D)), full((1, D))],
        out_specs=pl.BlockSpec((BN, D), lambda i: (i, 0)),
        out_shape=jax.ShapeDtypeStruct((N, D), jnp.float32),
    )(x, *parts, Wna, Wnb, bn1, Wn2, bn2, g_n, b_n)


def kernel(x, edge_index, edge_attr_raw, W_enc, b_enc, We1, be1, We2, be2,
           g_e, b_e, Wn1, bn1, Wn2, bn2, g_n, b_n):
    src = edge_index[0].astype(jnp.int32)
    dst = edge_index[1].astype(jnp.int32)
    zeros = jnp.zeros((NP, D), jnp.float32)

    Wea = We1[:D].astype(jnp.bfloat16)
    Wes = We1[D:2 * D].astype(jnp.bfloat16)
    Wed = We1[2 * D:].astype(jnp.bfloat16)
    We2b = We2.astype(jnp.bfloat16)
    Wna, Wnb = Wn1[:D], Wn1[D:]
    b_enc2 = b_enc.reshape(1, D)
    be1_2, be2_2 = be1.reshape(1, H), be2.reshape(1, D)
    ge2, be_2 = g_e.reshape(1, D), b_e.reshape(1, D)
    bn1_2, bn2_2 = bn1.reshape(1, H), bn2.reshape(1, D)
    gn2, bn_2 = g_n.reshape(1, D), b_n.reshape(1, D)

    splits = [(0, 102400), (102400, 102400), (204800, 115200)]
    halves = [
        dict(e0=e0, src=src[e0:e0 + ne], dst4=dst[e0:e0 + ne],
             ea=edge_attr_raw[e0:e0 + ne],
             epw=ne // NW, nwin=ne // NW // GW)
        for e0, ne in splits
    ]
    for step in range(STEPS):
        for hv in halves:
            hv["xs"], hv["xd"] = _sc_gather(x, hv["src"], hv["dst4"],
                                            hv["epw"], hv["nwin"])
        for hv in halves:
            hv["ea"] = _tc_edge(step == 0, hv["ea"], hv["xs"], hv["xd"],
                                W_enc, b_enc2, Wea, Wes, Wed, be1_2,
                                We2b, be2_2, ge2, be_2)
        for hv in halves:
            hv["p"] = _sc_scatter(hv["ea"], dst, zeros,
                                  hv["epw"], hv["nwin"], hv["e0"])
        parts = [hv["p"][c, :N] for hv in halves for c in range(NC)]
        x = _tc_node(x, parts, Wna, Wnb, bn1_2, Wn2, bn2_2, gn2, bn_2)
    return x
